# baseline (speedup 1.0000x reference)
import functools

import jax
import jax.numpy as jnp
from jax import lax
from jax.experimental import pallas as pl
from jax.experimental.pallas import tpu as pltpu
from jax.experimental.pallas import tpu_sc as plsc

HIDDEN = 64
PADW = 128
SEQ_LEN = 200
BATCH = 4096
TAB_ROWS = 1000008
TBLK = 1024
NC, NS = 2, 16
NW = NC * NS
BPW = BATCH // NW
SPLIT = 104
CHUNK = (SPLIT, SEQ_LEN - SPLIT)
NCHUNK = 2 * BPW
LANES = 16
NBUF = 4


def _build():
    mesh = plsc.VectorSubcoreMesh(core_axis_name="c", subcore_axis_name="s")

    @functools.partial(
        pl.kernel,
        out_type=jax.ShapeDtypeStruct((BATCH, SEQ_LEN, PADW), jnp.float32),
        mesh=mesh,
        scratch_types=[
            pltpu.VMEM((BPW, SEQ_LEN), jnp.int32),
            pltpu.VMEM((NBUF, SPLIT, PADW), jnp.float32),
            pltpu.VMEM((SEQ_LEN, HIDDEN), jnp.float32),
            [pltpu.SemaphoreType.DMA] * NBUF,
            [pltpu.SemaphoreType.DMA] * NBUF,
        ],
        compiler_params=pltpu.CompilerParams(use_tc_tiling_on_sc=False),
    )
    def k(idx_hbm, table_hbm, pos_hbm, out_hbm, idx_v, bufs, pos_v,
          gsems, ssems):
        wid = lax.axis_index("s") * NC + lax.axis_index("c")
        base = wid * BPW
        pltpu.sync_copy(pos_hbm, pos_v)
        pltpu.sync_copy(idx_hbm.at[pl.ds(base, BPW)], idx_v)

        def issue_gather(c, k_static):
            part = k_static % 2
            rows = CHUNK[part]
            seq = lax.div(c, 2)
            iv = idx_v.at[seq, pl.ds(part * SPLIT, rows)]
            pltpu.async_copy(table_hbm.at[iv], bufs.at[k_static, pl.ds(0, rows)],
                             gsems[k_static])

        def wait_gather(k_static):
            rows = CHUNK[k_static % 2]
            pltpu.make_async_copy(table_hbm.at[pl.ds(0, rows)],
                                  bufs.at[k_static, pl.ds(0, rows)],
                                  gsems[k_static]).wait()

        def issue_store(c, k_static):
            part = k_static % 2
            rows = CHUNK[part]
            seq = lax.div(c, 2)
            pltpu.async_copy(bufs.at[k_static, pl.ds(0, rows)],
                             out_hbm.at[base + seq, pl.ds(part * SPLIT, rows)],
                             ssems[k_static])

        def wait_store(k_static):
            rows = CHUNK[k_static % 2]
            pltpu.make_async_copy(bufs.at[k_static, pl.ds(0, rows)],
                                  out_hbm.at[0, pl.ds(0, rows)],
                                  ssems[k_static]).wait()

        issue_gather(0, 0)
        issue_gather(1, 1)

        @pl.loop(0, NCHUNK // NBUF)
        def _grp(j):
            for kk in range(NBUF):
                c = j * NBUF + kk
                part = kk % 2
                rows = CHUNK[part]
                buf = bufs.at[kk]
                wait_gather(kk)

                k2 = (kk + 2) % NBUF
                if kk < 2:
                    @pl.when(j > 0)
                    def _():
                        wait_store(k2)
                    issue_gather(c + 2, k2)
                else:
                    wait_store(k2)

                    @pl.when(j < NCHUNK // NBUF - 1)
                    def _():
                        issue_gather(c + 2, k2)

                @pl.loop(0, rows, unroll=8)
                def _row(r):
                    for d in range(HIDDEN // LANES):
                        sl = pl.ds(d * LANES, LANES)
                        buf[r, sl] = buf[r, sl] + pos_v[part * SPLIT + r, sl]

                issue_store(c, kk)

        wait_store(NBUF - 2)
        wait_store(NBUF - 1)

    return k


_KERNEL = _build()


def _pad_body(in_ref, out_ref):
    ii = lax.broadcasted_iota(jnp.int32, (HIDDEN, HIDDEN), 0)
    jj = lax.broadcasted_iota(jnp.int32, (HIDDEN, HIDDEN), 1)
    eye = (ii == jj).astype(jnp.float32)
    out_ref[:, 0:HIDDEN] = lax.dot_general(
        in_ref[...], eye, (((0,), (0,)), ((), ())),
        preferred_element_type=jnp.float32)


_PAD = pl.pallas_call(
    _pad_body,
    out_shape=jax.ShapeDtypeStruct((TAB_ROWS, PADW), jnp.float32),
    grid=((TAB_ROWS + TBLK - 1) // TBLK,),
    in_specs=[pl.BlockSpec((HIDDEN, TBLK), lambda i: (0, i))],
    out_specs=pl.BlockSpec((TBLK, PADW), lambda i: (i, 0)),
)


def kernel(item_seq, ID_embeddings, positional_embeddings):
    tab128 = _PAD(ID_embeddings.T)
    out = _KERNEL(item_seq, tab128, positional_embeddings)
    return out[:, :, :HIDDEN]


# speedup vs baseline: 1.1956x; 1.1956x over previous
import functools

import jax
import jax.numpy as jnp
from jax import lax
from jax.experimental import pallas as pl
from jax.experimental.pallas import tpu as pltpu
from jax.experimental.pallas import tpu_sc as plsc

HIDDEN = 64
PADW = 128
SEQ_LEN = 200
BATCH = 4096
TAB_ROWS = 1000001
NC, NS = 2, 16
NW = NC * NS
BPW = BATCH // NW
SPLIT = 104
CHUNK = (SPLIT, SEQ_LEN - SPLIT)
NCHUNK = 2 * BPW
LANES = 16
NBUF = 4


def _build():
    mesh = plsc.VectorSubcoreMesh(core_axis_name="c", subcore_axis_name="s")

    @functools.partial(
        pl.kernel,
        out_type=jax.ShapeDtypeStruct((BATCH, SEQ_LEN, PADW), jnp.float32),
        mesh=mesh,
        scratch_types=[
            pltpu.VMEM((BPW, SEQ_LEN), jnp.int32),
            pltpu.VMEM((NBUF, SPLIT, PADW), jnp.float32),
            pltpu.VMEM((SEQ_LEN, HIDDEN), jnp.float32),
            [pltpu.SemaphoreType.DMA] * NBUF,
            [pltpu.SemaphoreType.DMA] * NBUF,
        ],
        compiler_params=pltpu.CompilerParams(use_tc_tiling_on_sc=False),
    )
    def k(idx_hbm, table_hbm, pos_hbm, out_hbm, idx_v, bufs, pos_v,
          gsems, ssems):
        wid = lax.axis_index("s") * NC + lax.axis_index("c")
        base = wid * BPW
        pltpu.sync_copy(pos_hbm, pos_v)
        pltpu.sync_copy(idx_hbm.at[pl.ds(base, BPW)], idx_v)

        def issue_gather(c, k_static):
            part = k_static % 2
            rows = CHUNK[part]
            seq = lax.div(c, 2)
            iv = idx_v.at[seq, pl.ds(part * SPLIT, rows)]
            pltpu.async_copy(table_hbm.at[iv], bufs.at[k_static, pl.ds(0, rows)],
                             gsems[k_static])

        def wait_gather(k_static):
            rows = CHUNK[k_static % 2]
            pltpu.make_async_copy(table_hbm.at[pl.ds(0, rows)],
                                  bufs.at[k_static, pl.ds(0, rows)],
                                  gsems[k_static]).wait()

        def issue_store(c, k_static):
            part = k_static % 2
            rows = CHUNK[part]
            seq = lax.div(c, 2)
            pltpu.async_copy(bufs.at[k_static, pl.ds(0, rows)],
                             out_hbm.at[base + seq, pl.ds(part * SPLIT, rows)],
                             ssems[k_static])

        def wait_store(k_static):
            rows = CHUNK[k_static % 2]
            pltpu.make_async_copy(bufs.at[k_static, pl.ds(0, rows)],
                                  out_hbm.at[0, pl.ds(0, rows)],
                                  ssems[k_static]).wait()

        issue_gather(0, 0)
        issue_gather(1, 1)

        @pl.loop(0, NCHUNK // NBUF)
        def _grp(j):
            for kk in range(NBUF):
                c = j * NBUF + kk
                part = kk % 2
                rows = CHUNK[part]
                buf = bufs.at[kk]
                wait_gather(kk)

                k2 = (kk + 2) % NBUF
                if kk < 2:
                    @pl.when(j > 0)
                    def _():
                        wait_store(k2)
                    issue_gather(c + 2, k2)
                else:
                    wait_store(k2)

                    @pl.when(j < NCHUNK // NBUF - 1)
                    def _():
                        issue_gather(c + 2, k2)

                @pl.loop(0, rows, unroll=8)
                def _row(r):
                    for d in range(HIDDEN // LANES):
                        sl = pl.ds(d * LANES, LANES)
                        buf[r, sl] = buf[r, sl] + pos_v[part * SPLIT + r, sl]

                issue_store(c, kk)

        wait_store(NBUF - 2)
        wait_store(NBUF - 1)

    return k


_KERNEL = _build()


def kernel(item_seq, ID_embeddings, positional_embeddings):
    tab128 = jnp.pad(ID_embeddings, ((0, 0), (0, PADW - HIDDEN)))
    out = _KERNEL(item_seq, tab128, positional_embeddings)
    return out[:, :, :HIDDEN]
